# baseline (speedup 1.0000x reference)
import functools

import jax
import jax.numpy as jnp
from jax import lax
from jax.experimental import pallas as pl
from jax.experimental.pallas import tpu as pltpu
from jax.experimental.pallas import tpu_sc as plsc

_LANES = 128


def _log_softmax_table_kernel(it_ref, u_ref, out_ref):
    it = it_ref[...]
    u = u_ref[...]
    utilT = lax.dot_general(it, u, (((1,), (1,)), ((), ())),
                            preferred_element_type=jnp.float32)
    I, Upad = utilT.shape
    m = jnp.max(utilT, axis=0, keepdims=True)
    e = jnp.exp(utilT - m)
    lse = m + jnp.log(jnp.sum(e, axis=0, keepdims=True))
    logpT = utilT - lse
    BS = out_ref.shape[0] // (Upad // _LANES)
    for j in range(Upad // _LANES):
        out_ref[pl.ds(j * BS, I), :] = logpT[:, j * _LANES:(j + 1) * _LANES]


@functools.cache
def _make_expand(U, I, B, BS):
    info = plsc.get_sparse_core_info()
    NC, NS = info.num_cores, info.num_subcores
    NW = NC * NS
    NJ = (U + _LANES - 1) // _LANES
    R = (I + NW - 1) // NW
    NV = I - (NW - 1) * R
    assert R % 8 == 0 and NV > 0 and BS >= NW * R
    SB = 512
    n_chunks = B // SB
    assert B % SB == 0 and SB % 16 == 0
    mesh = plsc.VectorSubcoreMesh(core_axis_name="c", subcore_axis_name="s")

    @functools.partial(
        pl.kernel, mesh=mesh,
        out_type=jax.ShapeDtypeStruct((I, B), jnp.float32),
        compiler_params=pltpu.CompilerParams(needs_layout_passes=False),
        scratch_types=[
            pltpu.VMEM((B,), jnp.int32),
            pltpu.VMEM((NJ * R, _LANES), jnp.float32),
            pltpu.VMEM((R, SB), jnp.float32),
            pltpu.VMEM((R, SB), jnp.float32),
            pltpu.SemaphoreType.DMA,
            pltpu.SemaphoreType.DMA,
            pltpu.SemaphoreType.DMA,
        ],
    )
    def expand(table_hbm, idx_hbm, out_hbm,
               idx_v, tbuf, obufA, obufB, semi, semwA, semwB):
        wid = lax.axis_index("s") * NC + lax.axis_index("c")
        row0 = pl.multiple_of(wid * R, R)
        last = wid == NW - 1
        cpi = pltpu.async_copy(idx_hbm, idx_v, semi)
        tcps = [pltpu.async_copy(table_hbm.at[pl.ds(j * BS + row0, R)],
                                 tbuf.at[pl.ds(j * R, R)], semi)
                for j in range(NJ)]
        cpi.wait()
        for cp in tcps:
            cp.wait()

        rvec = [jnp.full((16,), r, jnp.int32) for r in range(R)]

        def compute(obuf, c):
            def vec_body(k, carry):
                off = pl.multiple_of(c * SB + 16 * k, 16)
                u16 = idx_v[pl.ds(off, 16)]
                hi = lax.shift_right_logical(u16, 7)
                lo = lax.bitwise_and(u16, _LANES - 1)
                base = hi * R
                for r in range(R):
                    obuf[r, pl.ds(pl.multiple_of(16 * k, 16), 16)] = (
                        plsc.load_gather(tbuf, [base + rvec[r], lo]))
                return carry

            lax.fori_loop(0, SB // 16, vec_body, 0)

        def fire(obuf, sem, c):
            col = pl.multiple_of(c * SB, SB)

            @pl.when(jnp.logical_not(last))
            def _():
                pltpu.async_copy(
                    obuf, out_hbm.at[pl.ds(row0, R), pl.ds(col, SB)], sem)

            @pl.when(last)
            def _():
                pltpu.async_copy(
                    obuf.at[pl.ds(0, NV)],
                    out_hbm.at[pl.ds(row0, NV), pl.ds(col, SB)], sem)

        def drain(obuf, sem):
            @pl.when(jnp.logical_not(last))
            def _():
                pltpu.make_async_copy(
                    out_hbm.at[pl.ds(0, R), pl.ds(0, SB)], obuf, sem).wait()

            @pl.when(last)
            def _():
                pltpu.make_async_copy(
                    out_hbm.at[pl.ds(0, NV), pl.ds(0, SB)],
                    obuf.at[pl.ds(0, NV)], sem).wait()

        def chunk_pair(p, carry):
            c0 = 2 * p

            @pl.when(p > 0)
            def _():
                drain(obufA, semwA)

            compute(obufA, c0)
            fire(obufA, semwA, c0)

            @pl.when(p > 0)
            def _():
                drain(obufB, semwB)

            compute(obufB, c0 + 1)
            fire(obufB, semwB, c0 + 1)
            return carry

        lax.fori_loop(0, n_chunks // 2, chunk_pair, 0)
        drain(obufA, semwA)
        drain(obufB, semwB)

    return expand


def kernel(user_latent_value, item_latent_value, user_idx):
    S, U, D = user_latent_value.shape
    I = item_latent_value.shape[1]
    B = user_idx.shape[0]
    Upad = (U + _LANES - 1) // _LANES * _LANES
    NW = 32
    BS = ((I + NW - 1) // NW) * NW
    u2 = user_latent_value.reshape(U, D)
    u2 = jnp.pad(u2, ((0, Upad - U), (0, 0)))
    it2 = item_latent_value.reshape(I, D)
    table = pl.pallas_call(
        _log_softmax_table_kernel,
        out_shape=jax.ShapeDtypeStruct(((Upad // _LANES) * BS, _LANES),
                                       jnp.float32),
    )(it2, u2)
    out2 = _make_expand(U, I, B, BS)(table, user_idx.astype(jnp.int32))
    return jnp.transpose(out2)[None]
